# Initial kernel scaffold; baseline (speedup 1.0000x reference)
import jax
import jax.numpy as jnp
from jax.experimental import pallas as pl


def kernel(pairs):
    raise NotImplementedError("write your pallas kernel here")


# speedup vs baseline: 44.1644x; 44.1644x over previous
import jax
import jax.numpy as jnp
from jax.experimental import pallas as pl

_NUM_LANDSCAPES = 5
_RESOLUTION = 1024
_ROWS_PER_STEP = 8


def _insert(accs, v):
    out = []
    for a in accs:
        hi = jnp.maximum(a, v)
        v = jnp.minimum(a, v)
        out.append(hi)
    return out


def _landscape_body(pairs_ref, out_ref):
    n = pairs_ref.shape[0]
    birth = pairs_ref[:, 0:1]
    death = pairs_ref[:, 1:2]
    min_b = jnp.min(birth)
    max_d = jnp.max(death)
    step = (max_d - min_b) / jnp.float32(_RESOLUTION - 1)
    lane = jax.lax.broadcasted_iota(jnp.int32, (1, _RESOLUTION), 1)
    t = min_b + step * lane.astype(jnp.float32)

    def body(i, accs):
        blk = pairs_ref[pl.ds(i * _ROWS_PER_STEP, _ROWS_PER_STEP), :]
        b = blk[:, 0:1]
        d = blk[:, 1:2]
        v = jnp.minimum(t - b, d - t)
        return tuple(_insert(accs, v))

    zero = jnp.zeros((_ROWS_PER_STEP, _RESOLUTION), jnp.float32)
    accs = jax.lax.fori_loop(0, n // _ROWS_PER_STEP, body,
                             (zero, zero, zero, zero, zero))
    accs = list(accs)

    for shift in (4, 2, 1):
        rolled = [jnp.roll(a, -shift, axis=0) for a in accs]
        for r in rolled:
            accs = _insert(accs, r)

    rows = [a[0:1, :] for a in accs]
    rows.append(jnp.zeros((8 - _NUM_LANDSCAPES, _RESOLUTION), jnp.float32))
    out_ref[:, :] = jnp.concatenate(rows, axis=0)


def kernel(pairs):
    out = pl.pallas_call(
        _landscape_body,
        out_shape=jax.ShapeDtypeStruct((8, _RESOLUTION), jnp.float32),
    )(pairs)
    return out[:_NUM_LANDSCAPES]


# speedup vs baseline: 74.9131x; 1.6962x over previous
import jax
import jax.numpy as jnp
from jax.experimental import pallas as pl

_NUM_LANDSCAPES = 5
_RESOLUTION = 1024
_ROWS_PER_STEP = 32


def _insert(accs, v):
    out = []
    for a in accs:
        hi = jnp.maximum(a, v)
        v = jnp.minimum(a, v)
        out.append(hi)
    return out


def _landscape_body(pairs_ref, out_ref):
    n = pairs_ref.shape[0]
    birth = pairs_ref[:, 0:1]
    death = pairs_ref[:, 1:2]
    min_b = jnp.min(birth)
    max_d = jnp.max(death)
    step = (max_d - min_b) / jnp.float32(_RESOLUTION - 1)
    lane = jax.lax.broadcasted_iota(jnp.int32, (1, _RESOLUTION), 1)
    t = min_b + step * lane.astype(jnp.float32)

    def body(i, accs):
        blk = pairs_ref[pl.ds(i * _ROWS_PER_STEP, _ROWS_PER_STEP), :]
        b = blk[:, 0:1]
        d = blk[:, 1:2]
        v = jnp.minimum(t - b, d - t)
        return tuple(_insert(accs, v))

    zero = jnp.zeros((_ROWS_PER_STEP, _RESOLUTION), jnp.float32)
    accs = jax.lax.fori_loop(0, n // _ROWS_PER_STEP, body,
                             (zero, zero, zero, zero, zero))
    accs = list(accs)

    for shift in (16, 8, 4, 2, 1):
        rolled = [jnp.roll(a, -shift, axis=0) for a in accs]
        for r in rolled:
            accs = _insert(accs, r)

    rows = [a[0:1, :] for a in accs]
    rows.append(jnp.zeros((8 - _NUM_LANDSCAPES, _RESOLUTION), jnp.float32))
    out_ref[:, :] = jnp.concatenate(rows, axis=0)


def kernel(pairs):
    out = pl.pallas_call(
        _landscape_body,
        out_shape=jax.ShapeDtypeStruct((8, _RESOLUTION), jnp.float32),
    )(pairs)
    return out[:_NUM_LANDSCAPES]


# speedup vs baseline: 86.3881x; 1.1532x over previous
import jax
import jax.numpy as jnp
from jax.experimental import pallas as pl

_NUM_LANDSCAPES = 5
_RESOLUTION = 1024
_ROWS_PER_STEP = 64


def _insert(accs, v):
    out = []
    for a in accs:
        hi = jnp.maximum(a, v)
        v = jnp.minimum(a, v)
        out.append(hi)
    return out


def _landscape_body(pairs_ref, out_ref):
    n = pairs_ref.shape[0]
    birth = pairs_ref[:, 0:1]
    death = pairs_ref[:, 1:2]
    min_b = jnp.min(birth)
    max_d = jnp.max(death)
    step = (max_d - min_b) / jnp.float32(_RESOLUTION - 1)
    lane = jax.lax.broadcasted_iota(jnp.int32, (1, _RESOLUTION), 1)
    t = min_b + step * lane.astype(jnp.float32)

    def body(i, accs):
        blk = pairs_ref[pl.ds(i * _ROWS_PER_STEP, _ROWS_PER_STEP), :]
        b = blk[:, 0:1]
        d = blk[:, 1:2]
        v = jnp.minimum(t - b, d - t)
        return tuple(_insert(accs, v))

    zero = jnp.zeros((_ROWS_PER_STEP, _RESOLUTION), jnp.float32)
    accs = jax.lax.fori_loop(0, n // _ROWS_PER_STEP, body,
                             (zero, zero, zero, zero, zero))
    accs = list(accs)

    for shift in (32, 16, 8, 4, 2, 1):
        rolled = [jnp.roll(a, -shift, axis=0) for a in accs]
        for r in rolled:
            accs = _insert(accs, r)

    rows = [a[0:1, :] for a in accs]
    rows.append(jnp.zeros((8 - _NUM_LANDSCAPES, _RESOLUTION), jnp.float32))
    out_ref[:, :] = jnp.concatenate(rows, axis=0)


def kernel(pairs):
    out = pl.pallas_call(
        _landscape_body,
        out_shape=jax.ShapeDtypeStruct((8, _RESOLUTION), jnp.float32),
    )(pairs)
    return out[:_NUM_LANDSCAPES]
